# baseline (speedup 1.0000x reference)
import jax
import jax.numpy as jnp
from jax.experimental import pallas as pl
from jax.experimental.pallas import tpu as pltpu

NV = 100000
NE = 50000
H = 128
NINC = 3 * NE


def _mm1_body(x_ref, w_ref, b_ref, wt_ref, y_ref, xw_ref):
    x = x_ref[...]
    wt = wt_ref[...]
    y = jnp.dot(x, w_ref[...], preferred_element_type=jnp.float32) + b_ref[...]
    y_ref[...] = jnp.maximum(y, 0.0) * wt
    xw_ref[...] = x * wt


def _mm2_body(x_ref, w_ref, b_ref, wt_ref, y_ref):
    x = x_ref[...]
    y = jnp.dot(x, w_ref[...], preferred_element_type=jnp.float32) + b_ref[...]
    y_ref[...] = jnp.maximum(y, 0.0) * wt_ref[...]


def _mm1(v, W, b, wt, blk=1000):
    n = v.shape[0]
    grid = (n // blk,)
    return pl.pallas_call(
        _mm1_body,
        grid=grid,
        in_specs=[
            pl.BlockSpec((blk, H), lambda i: (i, 0)),
            pl.BlockSpec((H, H), lambda i: (0, 0)),
            pl.BlockSpec((1, H), lambda i: (0, 0)),
            pl.BlockSpec((blk, 1), lambda i: (i, 0)),
        ],
        out_specs=[
            pl.BlockSpec((blk, H), lambda i: (i, 0)),
            pl.BlockSpec((blk, H), lambda i: (i, 0)),
        ],
        out_shape=[
            jax.ShapeDtypeStruct((n, H), jnp.float32),
            jax.ShapeDtypeStruct((n, H), jnp.float32),
        ],
    )(v, W, b.reshape(1, H), wt)


def _mm2(x, W, b, wt, blk=1000):
    n = x.shape[0]
    grid = (n // blk,)
    return pl.pallas_call(
        _mm2_body,
        grid=grid,
        in_specs=[
            pl.BlockSpec((blk, H), lambda i: (i, 0)),
            pl.BlockSpec((H, H), lambda i: (0, 0)),
            pl.BlockSpec((1, H), lambda i: (0, 0)),
            pl.BlockSpec((blk, 1), lambda i: (i, 0)),
        ],
        out_specs=pl.BlockSpec((blk, H), lambda i: (i, 0)),
        out_shape=jax.ShapeDtypeStruct((n, H), jnp.float32),
    )(x, W, b.reshape(1, H), wt)


def kernel(v, e, batch_idx, W_v2e, W_e2v, b_v, b_e, paper_author, eidx, vidx,
           v_weight, e_weight, v_reg_weight, e_reg_weight, e_reg_sum, v_reg_sum):
    ve_w, v_base = _mm1(v, W_v2e, b_v, v_weight)
    ve_g = ve_w[paper_author[:, 0]] * v_reg_weight
    e_new = e.at[eidx].add(ve_g)
    e_new = e_new / e_reg_sum
    ev_w = _mm2(e_new, W_e2v, b_e, e_weight)
    ev_vtx = ev_w[paper_author[:, 1]] * e_reg_weight
    v_out = v_base.at[vidx].add(ev_vtx)
    v_out = v_out / v_reg_sum
    return (v_out, e_new)


# speedup vs baseline: 1.4694x; 1.4694x over previous
import functools

import jax
import jax.numpy as jnp
from jax import lax
from jax.experimental import pallas as pl
from jax.experimental.pallas import tpu as pltpu
from jax.experimental.pallas import tpu_sc as plsc

NV = 100000
NE = 50000
H = 128
NINC = 3 * NE

NUM_CORES = 2
NUM_SUBCORES = 16
LANES = 16

CHUNK = 512
SUB = 128
NCHUNKS = 19
PER_TILE = CHUNK * NCHUNKS
NINC_PAD = PER_TILE * NUM_SUBCORES
ZROWS = 625


def _mm1_body(x_ref, w_ref, b_ref, wt_ref, y_ref):
    x = x_ref[...]
    y = jnp.dot(x, w_ref[...], preferred_element_type=jnp.float32) + b_ref[...]
    y_ref[...] = jnp.maximum(y, 0.0) * wt_ref[...]


def _linear(v, W, b, wt, blk=1000):
    n = v.shape[0]
    return pl.pallas_call(
        _mm1_body,
        grid=(n // blk,),
        in_specs=[
            pl.BlockSpec((blk, H), lambda i: (i, 0)),
            pl.BlockSpec((H, H), lambda i: (0, 0)),
            pl.BlockSpec((1, H), lambda i: (0, 0)),
            pl.BlockSpec((blk, 1), lambda i: (i, 0)),
        ],
        out_specs=pl.BlockSpec((blk, H), lambda i: (i, 0)),
        out_shape=jax.ShapeDtypeStruct((n, H), jnp.float32),
    )(v, W, b.reshape(1, H), wt)


def _stage3_body(e_ref, sc_ref, ers_ref, w_ref, b_ref, ew_ref, enew_ref, evw_ref):
    e_new = (e_ref[...] + sc_ref[...]) / ers_ref[...]
    enew_ref[...] = e_new
    y = jnp.dot(e_new, w_ref[...], preferred_element_type=jnp.float32) + b_ref[...]
    evw_ref[...] = jnp.maximum(y, 0.0) * ew_ref[...]


def _stage3(e, scat, ers, W, b, ew, blk=1000):
    n = e.shape[0]
    return pl.pallas_call(
        _stage3_body,
        grid=(n // blk,),
        in_specs=[
            pl.BlockSpec((blk, H), lambda i: (i, 0)),
            pl.BlockSpec((blk, H), lambda i: (i, 0)),
            pl.BlockSpec((blk, 1), lambda i: (i, 0)),
            pl.BlockSpec((H, H), lambda i: (0, 0)),
            pl.BlockSpec((1, H), lambda i: (0, 0)),
            pl.BlockSpec((blk, 1), lambda i: (i, 0)),
        ],
        out_specs=[
            pl.BlockSpec((blk, H), lambda i: (i, 0)),
            pl.BlockSpec((blk, H), lambda i: (i, 0)),
        ],
        out_shape=[
            jax.ShapeDtypeStruct((n, H), jnp.float32),
            jax.ShapeDtypeStruct((n, H), jnp.float32),
        ],
    )(e, scat, ers, W, b.reshape(1, H), ew)


def _stage5_body(v_ref, vw_ref, sc_ref, vrs_ref, out_ref):
    out_ref[...] = (v_ref[...] * vw_ref[...] + sc_ref[...]) / vrs_ref[...]


def _stage5(v, vw, scat, vrs, blk=1000):
    n = v.shape[0]
    return pl.pallas_call(
        _stage5_body,
        grid=(n // blk,),
        in_specs=[
            pl.BlockSpec((blk, H), lambda i: (i, 0)),
            pl.BlockSpec((blk, 1), lambda i: (i, 0)),
            pl.BlockSpec((blk, H), lambda i: (i, 0)),
            pl.BlockSpec((blk, 1), lambda i: (i, 0)),
        ],
        out_specs=pl.BlockSpec((blk, H), lambda i: (i, 0)),
        out_shape=jax.ShapeDtypeStruct((n, H), jnp.float32),
    )(v, vw, scat, vrs)


def _make_sc_scatter(n_src, n_dst, cb, wd):
    nb_per_sc = cb // NUM_CORES
    rpt = n_dst // NUM_SUBCORES
    nz = rpt // ZROWS
    mesh = plsc.VectorSubcoreMesh(
        core_axis_name="c", subcore_axis_name="s",
        num_cores=NUM_CORES, num_subcores=NUM_SUBCORES)

    @functools.partial(
        pl.kernel,
        mesh=mesh,
        compiler_params=pltpu.CompilerParams(use_tc_tiling_on_sc=False),
        out_type=jax.ShapeDtypeStruct((n_dst, cb, wd), jnp.float32),
        scratch_types=[
            pltpu.VMEM((CHUNK,), jnp.int32),
            pltpu.VMEM((CHUNK // SUB, SUB), jnp.int32),
            pltpu.VMEM((CHUNK,), jnp.float32),
            pltpu.VMEM((CHUNK, wd), jnp.float32),
            pltpu.VMEM((ZROWS, wd), jnp.float32),
            pltpu.VMEM_SHARED((n_dst, wd), jnp.float32),
            pltpu.SemaphoreType.DMA,
        ],
    )
    def k(tab, gidx, sidx, w, out, gix_v, six_v, w_v, rows_v, zeros_v, acc, sem):
        c = lax.axis_index("c")
        s = lax.axis_index("s")

        z16 = jnp.zeros((LANES,), jnp.float32)
        def zfill(r, _):
            for l in range(wd // LANES):
                zeros_v[r, pl.ds(l * LANES, LANES)] = z16
            return 0
        lax.fori_loop(0, ZROWS, zfill, 0)

        for j in range(nb_per_sc):
            b = c * nb_per_sc + j

            for z in range(nz):
                pltpu.sync_copy(zeros_v, acc.at[pl.ds((s * nz + z) * ZROWS, ZROWS)])
            plsc.subcore_barrier()

            def chunk_body(ch, _):
                base = s * PER_TILE + ch * CHUNK
                row0 = s * (PER_TILE // SUB) + ch * (CHUNK // SUB)
                pltpu.sync_copy(gidx.at[pl.ds(base, CHUNK)], gix_v)
                pltpu.sync_copy(sidx.at[pl.ds(row0, CHUNK // SUB)], six_v)
                pltpu.sync_copy(w.at[pl.ds(base, CHUNK)], w_v)
                for i in range(CHUNK // LANES):
                    sl = pl.ds(i * LANES, LANES)
                    gix_v[sl] = gix_v[sl] * cb + b
                copies = []
                for u in range(CHUNK // SUB):
                    copies.append(pltpu.async_copy(
                        tab.at[gix_v.at[pl.ds(u * SUB, SUB)]],
                        rows_v.at[pl.ds(u * SUB, SUB)], sem))
                for cp in copies:
                    cp.wait()
                def scale(g, _):
                    w16 = w_v[pl.ds(g * LANES, LANES)]
                    for jj in range(LANES):
                        r = g * LANES + jj
                        bw = lax.gather(
                            w16, jnp.full((LANES, 1), jj, jnp.int32),
                            lax.GatherDimensionNumbers(
                                offset_dims=(), collapsed_slice_dims=(0,),
                                start_index_map=(0,)),
                            (1,),
                            mode=lax.GatherScatterMode.PROMISE_IN_BOUNDS)
                        for l in range(wd // LANES):
                            sl = pl.ds(l * LANES, LANES)
                            rows_v[r, sl] = rows_v[r, sl] * bw
                    return 0
                lax.fori_loop(0, CHUNK // LANES, scale, 0)
                for u in range(CHUNK // SUB):
                    pltpu.sync_copy(rows_v.at[pl.ds(u * SUB, SUB)],
                                    acc.at[six_v.at[u]], add=True)
                return 0
            lax.fori_loop(0, NCHUNKS, chunk_body, 0)
            plsc.subcore_barrier()

            for z in range(nz):
                r0 = (s * nz + z) * ZROWS
                pltpu.sync_copy(acc.at[pl.ds(r0, ZROWS)],
                                out.at[pl.ds(r0, ZROWS), b])
            plsc.subcore_barrier()

    return k


def kernel(v, e, batch_idx, W_v2e, W_e2v, b_v, b_e, paper_author, eidx, vidx,
           v_weight, e_weight, v_reg_weight, e_reg_weight, e_reg_sum, v_reg_sum):
    pad = NINC_PAD - NINC
    pad_i = jnp.arange(pad, dtype=jnp.int32)

    def prep(g, sidx_, wt, n_src, n_dst):
        g = jnp.concatenate([g.astype(jnp.int32), pad_i % n_src])
        si = jnp.concatenate([sidx_.astype(jnp.int32), pad_i % n_dst])
        w = jnp.concatenate([wt.reshape(-1), jnp.zeros((pad,), jnp.float32)])
        return g, si.reshape(NINC_PAD // SUB, SUB), w

    ve_w = _linear(v, W_v2e, b_v, v_weight)

    g_e, s_e, w_e = prep(paper_author[:, 0], eidx, v_reg_weight, NV, NE)
    scat_e = _make_sc_scatter(NV, NE, 8, 16)(
        ve_w.reshape(NV * 8, 16), g_e, s_e, w_e)

    e_new, ev_w = _stage3(e, scat_e.reshape(NE, H), e_reg_sum, W_e2v, b_e,
                          e_weight)

    g_v, s_v, w_v = prep(paper_author[:, 1], vidx, e_reg_weight, NE, NV)
    scat_v = _make_sc_scatter(NE, NV, 8, 16)(
        ev_w.reshape(NE * 8, 16), g_v, s_v, w_v)

    v_out = _stage5(v, v_weight, scat_v.reshape(NV, H), v_reg_sum)
    return (v_out, e_new)


# speedup vs baseline: 1.6374x; 1.1144x over previous
import functools

import jax
import jax.numpy as jnp
from jax import lax
from jax.experimental import pallas as pl
from jax.experimental.pallas import tpu as pltpu
from jax.experimental.pallas import tpu_sc as plsc

NV = 100000
NE = 50000
H = 128
NINC = 3 * NE

NUM_CORES = 2
NUM_SUBCORES = 16
LANES = 16

CHUNK = 128
SUB = 128
NCHUNKS = 76
PER_TILE = CHUNK * NCHUNKS
NINC_PAD = PER_TILE * NUM_SUBCORES
ZROWS = 625


def _mm1_body(x_ref, w_ref, b_ref, wt_ref, y_ref):
    x = x_ref[...]
    y = jnp.dot(x, w_ref[...], preferred_element_type=jnp.float32) + b_ref[...]
    y_ref[...] = jnp.maximum(y, 0.0) * wt_ref[...]


def _linear(v, W, b, wt, blk=1000):
    n = v.shape[0]
    return pl.pallas_call(
        _mm1_body,
        grid=(n // blk,),
        in_specs=[
            pl.BlockSpec((blk, H), lambda i: (i, 0)),
            pl.BlockSpec((H, H), lambda i: (0, 0)),
            pl.BlockSpec((1, H), lambda i: (0, 0)),
            pl.BlockSpec((blk, 1), lambda i: (i, 0)),
        ],
        out_specs=pl.BlockSpec((blk, H), lambda i: (i, 0)),
        out_shape=jax.ShapeDtypeStruct((n, H), jnp.float32),
    )(v, W, b.reshape(1, H), wt)


def _stage3_body(e_ref, sc_ref, ers_ref, w_ref, b_ref, ew_ref, enew_ref, evw_ref):
    e_new = (e_ref[...] + sc_ref[...]) / ers_ref[...]
    enew_ref[...] = e_new
    y = jnp.dot(e_new, w_ref[...], preferred_element_type=jnp.float32) + b_ref[...]
    evw_ref[...] = jnp.maximum(y, 0.0) * ew_ref[...]


def _stage3(e, scat, ers, W, b, ew, blk=1000):
    n = e.shape[0]
    return pl.pallas_call(
        _stage3_body,
        grid=(n // blk,),
        in_specs=[
            pl.BlockSpec((blk, H), lambda i: (i, 0)),
            pl.BlockSpec((blk, H), lambda i: (i, 0)),
            pl.BlockSpec((blk, 1), lambda i: (i, 0)),
            pl.BlockSpec((H, H), lambda i: (0, 0)),
            pl.BlockSpec((1, H), lambda i: (0, 0)),
            pl.BlockSpec((blk, 1), lambda i: (i, 0)),
        ],
        out_specs=[
            pl.BlockSpec((blk, H), lambda i: (i, 0)),
            pl.BlockSpec((blk, H), lambda i: (i, 0)),
        ],
        out_shape=[
            jax.ShapeDtypeStruct((n, H), jnp.float32),
            jax.ShapeDtypeStruct((n, H), jnp.float32),
        ],
    )(e, scat, ers, W, b.reshape(1, H), ew)


def _stage5_body(v_ref, vw_ref, sc_ref, vrs_ref, out_ref):
    out_ref[...] = (v_ref[...] * vw_ref[...] + sc_ref[...]) / vrs_ref[...]


def _stage5(v, vw, scat, vrs, blk=1000):
    n = v.shape[0]
    return pl.pallas_call(
        _stage5_body,
        grid=(n // blk,),
        in_specs=[
            pl.BlockSpec((blk, H), lambda i: (i, 0)),
            pl.BlockSpec((blk, 1), lambda i: (i, 0)),
            pl.BlockSpec((blk, H), lambda i: (i, 0)),
            pl.BlockSpec((blk, 1), lambda i: (i, 0)),
        ],
        out_specs=pl.BlockSpec((blk, H), lambda i: (i, 0)),
        out_shape=jax.ShapeDtypeStruct((n, H), jnp.float32),
    )(v, vw, scat, vrs)


def _make_sc_scatter(n_src, n_dst, cb, wd):
    nb_per_sc = cb // NUM_CORES
    rpt = n_dst // NUM_SUBCORES
    nz = rpt // ZROWS
    mesh = plsc.VectorSubcoreMesh(
        core_axis_name="c", subcore_axis_name="s",
        num_cores=NUM_CORES, num_subcores=NUM_SUBCORES)

    @functools.partial(
        pl.kernel,
        mesh=mesh,
        compiler_params=pltpu.CompilerParams(use_tc_tiling_on_sc=False),
        out_type=jax.ShapeDtypeStruct((n_dst, cb, wd), jnp.float32),
        scratch_types=[
            [pltpu.VMEM((CHUNK,), jnp.int32) for _ in range(4)],
            [pltpu.VMEM((CHUNK // SUB, SUB), jnp.int32) for _ in range(4)],
            [pltpu.VMEM((CHUNK,), jnp.float32) for _ in range(4)],
            [pltpu.VMEM((CHUNK, wd), jnp.float32) for _ in range(2)],
            pltpu.VMEM((ZROWS, wd), jnp.float32),
            pltpu.VMEM_SHARED((n_dst, wd), jnp.float32),
            pltpu.SemaphoreType.DMA((4,)),
            pltpu.SemaphoreType.DMA((2,)),
            pltpu.SemaphoreType.DMA((2,)),
        ],
    )
    def k(tab, gidx, sidx, w, out, gixs, sixs, ws, rowss, zeros_v, acc,
          sem_i, sem_g, sem_s):
        c = lax.axis_index("c")
        s = lax.axis_index("s")

        def idx_copies(ch, b):
            base = s * PER_TILE + ch * CHUNK
            row0 = s * (PER_TILE // SUB) + ch * (CHUNK // SUB)
            return [
                (gidx.at[pl.ds(base, CHUNK)], gixs[b]),
                (sidx.at[pl.ds(row0, CHUNK // SUB)], sixs[b]),
                (w.at[pl.ds(base, CHUNK)], ws[b]),
            ]

        def idx_start(ch, b):
            for src, dst in idx_copies(ch, b):
                pltpu.async_copy(src, dst, sem_i.at[b])

        def idx_wait(ch, b):
            for src, dst in idx_copies(ch, b):
                pltpu.make_async_copy(src, dst, sem_i.at[b]).wait()

        def adjust(b, blk):
            for i in range(CHUNK // LANES):
                sl = pl.ds(i * LANES, LANES)
                gixs[b][sl] = gixs[b][sl] * cb + blk

        def gather_copies(b, rb):
            return [
                (tab.at[gixs[b].at[pl.ds(u * SUB, SUB)]],
                 rowss[rb].at[pl.ds(u * SUB, SUB)])
                for u in range(CHUNK // SUB)
            ]

        def gather_start(b, rb):
            for src, dst in gather_copies(b, rb):
                pltpu.async_copy(src, dst, sem_g.at[rb])

        def gather_wait(b, rb):
            for src, dst in gather_copies(b, rb):
                pltpu.make_async_copy(src, dst, sem_g.at[rb]).wait()

        def scale(b, rb):
            def body(g, _):
                w16 = ws[b][pl.ds(g * LANES, LANES)]
                for jj in range(LANES):
                    r = g * LANES + jj
                    bw = lax.gather(
                        w16, jnp.full((LANES, 1), jj, jnp.int32),
                        lax.GatherDimensionNumbers(
                            offset_dims=(), collapsed_slice_dims=(0,),
                            start_index_map=(0,)),
                        (1,),
                        mode=lax.GatherScatterMode.PROMISE_IN_BOUNDS)
                    for l in range(wd // LANES):
                        sl = pl.ds(l * LANES, LANES)
                        rowss[rb][r, sl] = rowss[rb][r, sl] * bw
                return 0
            lax.fori_loop(0, CHUNK // LANES, body, 0)

        def scat_copies(b, rb):
            return [
                (rowss[rb].at[pl.ds(u * SUB, SUB)], acc.at[sixs[b].at[u]])
                for u in range(CHUNK // SUB)
            ]

        def scat_start(b, rb):
            for src, dst in scat_copies(b, rb):
                pltpu.async_copy(src, dst, sem_s.at[rb], add=True)

        def scat_wait(b, rb):
            for src, dst in scat_copies(b, rb):
                pltpu.make_async_copy(src, dst, sem_s.at[rb]).wait()

        z16 = jnp.zeros((LANES,), jnp.float32)
        def zfill(r, _):
            for l in range(wd // LANES):
                zeros_v[r, pl.ds(l * LANES, LANES)] = z16
            return 0
        lax.fori_loop(0, ZROWS, zfill, 0)

        for j in range(nb_per_sc):
            blk = c * nb_per_sc + j

            for z in range(nz):
                pltpu.sync_copy(zeros_v, acc.at[pl.ds((s * nz + z) * ZROWS, ZROWS)])
            plsc.subcore_barrier()

            idx_start(0, 0)
            idx_wait(0, 0)
            adjust(0, blk)
            gather_start(0, 0)
            idx_start(1, 1)

            def pipe(i, _):
                for off in range(4):
                    ch = i * 4 + off
                    p4, p2 = off % 4, off % 2
                    n4, n2 = (off + 1) % 4, (off + 1) % 2

                    @pl.when(ch + 1 < NCHUNKS)
                    def _():
                        idx_wait(ch + 1, n4)
                        adjust(n4, blk)

                    @pl.when(ch >= 1)
                    def _():
                        scat_wait((off - 1) % 4, (off - 1) % 2)

                    @pl.when(ch + 1 < NCHUNKS)
                    def _():
                        gather_start(n4, n2)

                    gather_wait(p4, p2)

                    @pl.when(ch + 2 < NCHUNKS)
                    def _():
                        idx_start(ch + 2, (off + 2) % 4)

                    scale(p4, p2)
                    scat_start(p4, p2)
                return 0
            lax.fori_loop(0, NCHUNKS // 4, pipe, 0)
            scat_wait((NCHUNKS - 1) % 4, (NCHUNKS - 1) % 2)
            plsc.subcore_barrier()

            for z in range(nz):
                r0 = (s * nz + z) * ZROWS
                pltpu.sync_copy(acc.at[pl.ds(r0, ZROWS)],
                                out.at[pl.ds(r0, ZROWS), blk])
            plsc.subcore_barrier()

    return k


def kernel(v, e, batch_idx, W_v2e, W_e2v, b_v, b_e, paper_author, eidx, vidx,
           v_weight, e_weight, v_reg_weight, e_reg_weight, e_reg_sum, v_reg_sum):
    pad = NINC_PAD - NINC
    pad_i = jnp.arange(pad, dtype=jnp.int32)

    def prep(g, sidx_, wt, n_src, n_dst):
        g = jnp.concatenate([g.astype(jnp.int32), pad_i % n_src])
        si = jnp.concatenate([sidx_.astype(jnp.int32), pad_i % n_dst])
        w = jnp.concatenate([wt.reshape(-1), jnp.zeros((pad,), jnp.float32)])
        return g, si.reshape(NINC_PAD // SUB, SUB), w

    ve_w = _linear(v, W_v2e, b_v, v_weight)

    g_e, s_e, w_e = prep(paper_author[:, 0], eidx, v_reg_weight, NV, NE)
    scat_e = _make_sc_scatter(NV, NE, 8, 16)(
        ve_w.reshape(NV * 8, 16), g_e, s_e, w_e)

    e_new, ev_w = _stage3(e, scat_e.reshape(NE, H), e_reg_sum, W_e2v, b_e,
                          e_weight)

    g_v, s_v, w_v = prep(paper_author[:, 1], vidx, e_reg_weight, NE, NV)
    scat_v = _make_sc_scatter(NE, NV, 8, 16)(
        ev_w.reshape(NE * 8, 16), g_v, s_v, w_v)

    v_out = _stage5(v, v_weight, scat_v.reshape(NV, H), v_reg_sum)
    return (v_out, e_new)


# speedup vs baseline: 2.4884x; 1.5197x over previous
import functools

import jax
import jax.numpy as jnp
from jax import lax
from jax.experimental import pallas as pl
from jax.experimental.pallas import tpu as pltpu
from jax.experimental.pallas import tpu_sc as plsc

NV = 100000
NE = 50000
H = 128
NINC = 3 * NE

NUM_CORES = 2
NUM_SUBCORES = 16
LANES = 16

CHUNK = 128
SUB = 128
NCHUNKS = 76
PER_TILE = CHUNK * NCHUNKS
NINC_PAD = PER_TILE * NUM_SUBCORES
ZROWS = 625


def _mm1_body(x_ref, w_ref, b_ref, wt_ref, y_ref):
    x = x_ref[...]
    y = jnp.dot(x, w_ref[...], preferred_element_type=jnp.float32) + b_ref[...]
    y_ref[...] = jnp.maximum(y, 0.0) * wt_ref[...]


def _linear(v, W, b, wt, blk=1000):
    n = v.shape[0]
    return pl.pallas_call(
        _mm1_body,
        grid=(n // blk,),
        in_specs=[
            pl.BlockSpec((blk, H), lambda i: (i, 0)),
            pl.BlockSpec((H, H), lambda i: (0, 0)),
            pl.BlockSpec((1, H), lambda i: (0, 0)),
            pl.BlockSpec((blk, 1), lambda i: (i, 0)),
        ],
        out_specs=pl.BlockSpec((blk, H), lambda i: (i, 0)),
        out_shape=jax.ShapeDtypeStruct((n, H), jnp.float32),
    )(v, W, b.reshape(1, H), wt)


def _stage3_body(e_ref, sc_ref, ers_ref, w_ref, b_ref, ew_ref, enew_ref, evw_ref):
    e_new = (e_ref[...] + sc_ref[...]) / ers_ref[...]
    enew_ref[...] = e_new
    y = jnp.dot(e_new, w_ref[...], preferred_element_type=jnp.float32) + b_ref[...]
    evw_ref[...] = jnp.maximum(y, 0.0) * ew_ref[...]


def _stage3(e, scat, ers, W, b, ew, blk=1000):
    n = e.shape[0]
    return pl.pallas_call(
        _stage3_body,
        grid=(n // blk,),
        in_specs=[
            pl.BlockSpec((blk, H), lambda i: (i, 0)),
            pl.BlockSpec((blk, H), lambda i: (i, 0)),
            pl.BlockSpec((blk, 1), lambda i: (i, 0)),
            pl.BlockSpec((H, H), lambda i: (0, 0)),
            pl.BlockSpec((1, H), lambda i: (0, 0)),
            pl.BlockSpec((blk, 1), lambda i: (i, 0)),
        ],
        out_specs=[
            pl.BlockSpec((blk, H), lambda i: (i, 0)),
            pl.BlockSpec((blk, H), lambda i: (i, 0)),
        ],
        out_shape=[
            jax.ShapeDtypeStruct((n, H), jnp.float32),
            jax.ShapeDtypeStruct((n, H), jnp.float32),
        ],
    )(e, scat, ers, W, b.reshape(1, H), ew)


def _stage5_body(v_ref, vw_ref, sc_ref, vrs_ref, out_ref):
    out_ref[...] = (v_ref[...] * vw_ref[...] + sc_ref[...]) / vrs_ref[...]


def _stage5(v, vw, scat, vrs, blk=1000):
    n = v.shape[0]
    return pl.pallas_call(
        _stage5_body,
        grid=(n // blk,),
        in_specs=[
            pl.BlockSpec((blk, H), lambda i: (i, 0)),
            pl.BlockSpec((blk, 1), lambda i: (i, 0)),
            pl.BlockSpec((blk, H), lambda i: (i, 0)),
            pl.BlockSpec((blk, 1), lambda i: (i, 0)),
        ],
        out_specs=pl.BlockSpec((blk, H), lambda i: (i, 0)),
        out_shape=jax.ShapeDtypeStruct((n, H), jnp.float32),
    )(v, vw, scat, vrs)


def _make_sc_scatter(n_src, n_dst, cb, wd):
    nb_per_sc = cb // NUM_CORES
    rpt = n_dst // NUM_SUBCORES
    nz = rpt // ZROWS
    mesh = plsc.VectorSubcoreMesh(
        core_axis_name="c", subcore_axis_name="s",
        num_cores=NUM_CORES, num_subcores=NUM_SUBCORES)

    @functools.partial(
        pl.kernel,
        mesh=mesh,
        compiler_params=pltpu.CompilerParams(use_tc_tiling_on_sc=False),
        out_type=jax.ShapeDtypeStruct((n_dst, H), jnp.float32),
        scratch_types=[
            [pltpu.VMEM((CHUNK,), jnp.int32) for _ in range(4)],
            [pltpu.VMEM((CHUNK // SUB, SUB), jnp.int32) for _ in range(4)],
            [pltpu.VMEM((CHUNK,), jnp.float32) for _ in range(4)],
            [pltpu.VMEM((CHUNK, wd), jnp.float32) for _ in range(2)],
            pltpu.VMEM((ZROWS, wd), jnp.float32),
            pltpu.VMEM_SHARED((n_dst, wd), jnp.float32),
            pltpu.SemaphoreType.DMA((4,)),
            pltpu.SemaphoreType.DMA((2,)),
            pltpu.SemaphoreType.DMA((2,)),
        ],
    )
    def k(tab, gidx, sidx, w, out, gixs, sixs, ws, rowss, zeros_v, acc,
          sem_i, sem_g, sem_s):
        c = lax.axis_index("c")
        s = lax.axis_index("s")

        def idx_copies(ch, b):
            base = s * PER_TILE + ch * CHUNK
            row0 = s * (PER_TILE // SUB) + ch * (CHUNK // SUB)
            return [
                (gidx.at[pl.ds(base, CHUNK)], gixs[b]),
                (sidx.at[pl.ds(row0, CHUNK // SUB)], sixs[b]),
                (w.at[pl.ds(base, CHUNK)], ws[b]),
            ]

        def idx_start(ch, b):
            for src, dst in idx_copies(ch, b):
                pltpu.async_copy(src, dst, sem_i.at[b])

        def idx_wait(ch, b):
            for src, dst in idx_copies(ch, b):
                pltpu.make_async_copy(src, dst, sem_i.at[b]).wait()

        def adjust(b, blk):
            for i in range(CHUNK // LANES):
                sl = pl.ds(i * LANES, LANES)
                gixs[b][sl] = gixs[b][sl] * cb + blk

        def gather_copies(b, rb):
            return [
                (tab.at[gixs[b].at[pl.ds(u * SUB, SUB)]],
                 rowss[rb].at[pl.ds(u * SUB, SUB)])
                for u in range(CHUNK // SUB)
            ]

        def gather_start(b, rb):
            for src, dst in gather_copies(b, rb):
                pltpu.async_copy(src, dst, sem_g.at[rb])

        def gather_wait(b, rb):
            for src, dst in gather_copies(b, rb):
                pltpu.make_async_copy(src, dst, sem_g.at[rb]).wait()

        def scale(b, rb):
            def body(g, _):
                w16 = ws[b][pl.ds(g * LANES, LANES)]
                for jj in range(LANES):
                    r = g * LANES + jj
                    bw = lax.gather(
                        w16, jnp.full((LANES, 1), jj, jnp.int32),
                        lax.GatherDimensionNumbers(
                            offset_dims=(), collapsed_slice_dims=(0,),
                            start_index_map=(0,)),
                        (1,),
                        mode=lax.GatherScatterMode.PROMISE_IN_BOUNDS)
                    for l in range(wd // LANES):
                        sl = pl.ds(l * LANES, LANES)
                        rowss[rb][r, sl] = rowss[rb][r, sl] * bw
                return 0
            lax.fori_loop(0, CHUNK // LANES, body, 0)

        def scat_copies(b, rb):
            return [
                (rowss[rb].at[pl.ds(u * SUB, SUB)], acc.at[sixs[b].at[u]])
                for u in range(CHUNK // SUB)
            ]

        def scat_start(b, rb):
            for src, dst in scat_copies(b, rb):
                pltpu.async_copy(src, dst, sem_s.at[rb], add=True)

        def scat_wait(b, rb):
            for src, dst in scat_copies(b, rb):
                pltpu.make_async_copy(src, dst, sem_s.at[rb]).wait()

        z16 = jnp.zeros((LANES,), jnp.float32)
        def zfill(r, _):
            for l in range(wd // LANES):
                zeros_v[r, pl.ds(l * LANES, LANES)] = z16
            return 0
        lax.fori_loop(0, ZROWS, zfill, 0)

        for j in range(nb_per_sc):
            blk = c * nb_per_sc + j

            for z in range(nz):
                pltpu.sync_copy(zeros_v, acc.at[pl.ds((s * nz + z) * ZROWS, ZROWS)])
            plsc.subcore_barrier()

            idx_start(0, 0)
            idx_wait(0, 0)
            adjust(0, blk)
            gather_start(0, 0)
            idx_start(1, 1)

            def pipe(i, _):
                for off in range(4):
                    ch = i * 4 + off
                    p4, p2 = off % 4, off % 2
                    n4, n2 = (off + 1) % 4, (off + 1) % 2

                    @pl.when(ch + 1 < NCHUNKS)
                    def _():
                        idx_wait(ch + 1, n4)
                        adjust(n4, blk)

                    @pl.when(ch >= 1)
                    def _():
                        scat_wait((off - 1) % 4, (off - 1) % 2)

                    @pl.when(ch + 1 < NCHUNKS)
                    def _():
                        gather_start(n4, n2)

                    gather_wait(p4, p2)

                    @pl.when(ch + 2 < NCHUNKS)
                    def _():
                        idx_start(ch + 2, (off + 2) % 4)

                    scale(p4, p2)
                    scat_start(p4, p2)
                return 0
            lax.fori_loop(0, NCHUNKS // 4, pipe, 0)
            scat_wait((NCHUNKS - 1) % 4, (NCHUNKS - 1) % 2)
            plsc.subcore_barrier()

            for z in range(nz):
                r0 = (s * nz + z) * ZROWS
                pltpu.sync_copy(acc.at[pl.ds(r0, ZROWS)],
                                out.at[pl.ds(r0, ZROWS), pl.ds(blk * wd, wd)])
            plsc.subcore_barrier()

    return k


def kernel(v, e, batch_idx, W_v2e, W_e2v, b_v, b_e, paper_author, eidx, vidx,
           v_weight, e_weight, v_reg_weight, e_reg_weight, e_reg_sum, v_reg_sum):
    pad = NINC_PAD - NINC
    pad_i = jnp.arange(pad, dtype=jnp.int32)

    def prep(g, sidx_, wt, n_src, n_dst):
        g = jnp.concatenate([g.astype(jnp.int32), pad_i % n_src])
        si = jnp.concatenate([sidx_.astype(jnp.int32), pad_i % n_dst])
        w = jnp.concatenate([wt.reshape(-1), jnp.zeros((pad,), jnp.float32)])
        return g, si.reshape(NINC_PAD // SUB, SUB), w

    ve_w = _linear(v, W_v2e, b_v, v_weight)

    g_e, s_e, w_e = prep(paper_author[:, 0], eidx, v_reg_weight, NV, NE)
    scat_e = _make_sc_scatter(NV, NE, 8, 16)(
        ve_w.reshape(NV * 8, 16), g_e, s_e, w_e)

    e_new, ev_w = _stage3(e, scat_e, e_reg_sum, W_e2v, b_e, e_weight)

    g_v, s_v, w_v = prep(paper_author[:, 1], vidx, e_reg_weight, NE, NV)
    scat_v = _make_sc_scatter(NE, NV, 8, 16)(
        ev_w.reshape(NE * 8, 16), g_v, s_v, w_v)

    v_out = _stage5(v, v_weight, scat_v, v_reg_sum)
    return (v_out, e_new)
